# baseline (speedup 1.0000x reference)
import jax
import jax.numpy as jnp
from jax import lax
from jax.experimental import pallas as pl
from jax.experimental.pallas import tpu as pltpu
from jax.experimental.pallas import tpu_sc as plsc

N_FD = 50000
E_FD = 800000
L = 24
T = 2048
T_SP = 1024
FD_ITERS = 8
N_PIN = 3072
N_PIN_PAD = 4096

NTILE = 16
COLS = 128
ROWS = 400
N_PAD = ROWS * COLS
ROWS_T = ROWS // NTILE
SGROUPS = ROWS_T * (COLS // 16)
CHUNK = 1280
ECROWS = CHUNK // COLS
NCHUNK = 40
E_TILE = CHUNK * NCHUNK
E_PAD = NTILE * E_TILE
GROUPS = CHUNK // 16
RCHUNKS = 5
RCLEN = ROWS // RCHUNKS
DUMMY = N_PAD - 1


def _fd_body(xs_hbm, loads_hbm, epack_hbm, pins_hbm, zeros_hbm,
             idrows_hbm, ownrows_hbm, outx_hbm, outres_hbm, parts_hbm,
             x_cur, s_part, ebuf0, ebuf1, idrows, ownrows, diag_s, a_s,
             b_s, red_s, esem0, esem1, zsem, rsem, spmem_sum):
    c = lax.axis_index("c")
    t = lax.axis_index("s")
    r_off = t * ROWS_T
    ones16 = jnp.full((16,), 1.0, jnp.float32)

    def rc(g):
        return g // 8, (g % 8) * 16

    def start_chunk(k, eb, sem):
        pltpu.async_copy(epack_hbm.at[t * NCHUNK + k], eb, sem)

    def wait_chunk(k, eb, sem):
        pltpu.make_async_copy(epack_hbm.at[t * NCHUNK + k], eb, sem).wait()

    def compute_chunk(eb, with_q_times_x):
        @plsc.parallel_loop(0, GROUPS, unroll=4)
        def _(g):
            r, cb = rc(g)
            sl = pl.ds(cb, 16)
            v = eb[r, sl]
            qv = plsc.bitcast(eb[ECROWS + r, sl], jnp.float32)
            iv = v & 0xFFFF
            jv = lax.shift_right_logical(v, 16)
            ir, ic = iv >> 7, iv & 127
            jr, jc = jv >> 7, jv & 127
            if with_q_times_x:
                xj = plsc.load_gather(x_cur, [jr, jc])
                xi = plsc.load_gather(x_cur, [ir, ic])
                plsc.addupdate_scatter(s_part, [ir, ic], qv * xj)
                plsc.addupdate_scatter(s_part, [jr, jc], qv * xi)
            else:
                plsc.addupdate_scatter(s_part, [ir, ic], qv)
                plsc.addupdate_scatter(s_part, [jr, jc], qv)

    def start_zero():
        pltpu.async_copy(zeros_hbm, s_part, zsem)

    def scatter_pass(with_q_times_x):
        start_chunk(0, ebuf0, esem0)
        pltpu.make_async_copy(zeros_hbm, s_part, zsem).wait()

        def pair_body(kk, _):
            k0 = 2 * kk
            start_chunk(k0 + 1, ebuf1, esem1)
            wait_chunk(k0, ebuf0, esem0)
            compute_chunk(ebuf0, with_q_times_x)

            @pl.when(k0 + 2 < NCHUNK)
            def _():
                start_chunk(k0 + 2, ebuf0, esem0)

            wait_chunk(k0 + 1, ebuf1, esem1)
            compute_chunk(ebuf1, with_q_times_x)
            return _

        lax.fori_loop(0, NCHUNK // 2, pair_body, None)

    def reduce_to_slice():
        def wr_body(k, _):
            pltpu.async_copy(s_part.at[pl.ds(k * ROWS_T, ROWS_T)],
                             parts_hbm.at[c, k, t], rsem)
            return _

        def wrw_body(k, _):
            pltpu.make_async_copy(s_part.at[pl.ds(k * ROWS_T, ROWS_T)],
                                  parts_hbm.at[c, k, t], rsem).wait()
            return _

        def rd_body(k, _):
            pltpu.async_copy(parts_hbm.at[c, t, k],
                             s_part.at[pl.ds(k * ROWS_T, ROWS_T)], rsem)
            return _

        def rdw_body(k, _):
            pltpu.make_async_copy(parts_hbm.at[c, t, k],
                                  s_part.at[pl.ds(k * ROWS_T, ROWS_T)],
                                  rsem).wait()
            return _

        lax.fori_loop(0, NTILE, wr_body, None)
        lax.fori_loop(0, NTILE, wrw_body, None)
        plsc.subcore_barrier()
        lax.fori_loop(0, NTILE, rd_body, None)
        lax.fori_loop(0, NTILE, rdw_body, None)

        def sum_body(g, _):
            r, cb = rc(g)
            sl = pl.ds(cb, 16)
            acc = s_part[r, sl]
            for k in range(1, NTILE):
                acc = acc + s_part[k * ROWS_T + r, sl]
            red_s[r, sl] = acc
            return _

        lax.fori_loop(0, SGROUPS, sum_body, None)
        start_zero()

    pltpu.sync_copy(idrows_hbm, idrows)
    pltpu.sync_copy(ownrows_hbm.at[t], ownrows)

    pltpu.sync_copy(zeros_hbm, s_part)
    pltpu.sync_copy(pins_hbm.at[0], ebuf0.at[pl.ds(0, 16)])
    pltpu.sync_copy(pins_hbm.at[1], ebuf1.at[pl.ds(0, 16)])

    for eb in (ebuf0, ebuf1):
        def pin_body(g, _, eb=eb):
            r, cb = rc(g)
            pv = eb[r, pl.ds(cb, 16)]
            plsc.addupdate_scatter(s_part, [pv >> 7, pv & 127], ones16)
            return _

        lax.fori_loop(0, (N_PIN_PAD // 2) // 16, pin_body, None)

    def mask_body(g, _):
        r, cb = rc(g)
        a_s[r, pl.ds(cb, 16)] = 1.0 - jnp.minimum(
            s_part[r_off + r, pl.ds(cb, 16)], 1.0)
        return _

    lax.fori_loop(0, SGROUPS, mask_body, None)
    start_zero()

    scatter_pass(with_q_times_x=False)
    reduce_to_slice()

    def diag_body(g, _):
        r, cb = rc(g)
        sl = pl.ds(cb, 16)
        dv = red_s[r, sl] + 1e-6
        diag_s[r, sl] = dv
        a_s[r, sl] = a_s[r, sl] / dv
        return _

    lax.fori_loop(0, SGROUPS, diag_body, None)
    plsc.subcore_barrier()

    for slot in range(2):
        coord = 2 * c + slot

        @pl.when(coord <= 2)
        def _():
            pltpu.sync_copy(xs_hbm.at[coord], x_cur)
            pltpu.sync_copy(loads_hbm.at[coord, t], b_s)

            def b_body(g, _):
                r, cb = rc(g)
                sl = pl.ds(cb, 16)
                m = 1.0 - a_s[r, sl] * diag_s[r, sl]
                x0 = x_cur[r_off + r, sl]
                b_s[r, sl] = m * x0 + b_s[r, sl] * a_s[r, sl]
                return _

            lax.fori_loop(0, SGROUPS, b_body, None)
            plsc.subcore_barrier()

            def iter_body(p, _):
                scatter_pass(with_q_times_x=True)
                reduce_to_slice()

                def upd_body(g, _):
                    r, cb = rc(g)
                    sl = pl.ds(cb, 16)
                    red_s[r, sl] = b_s[r, sl] + a_s[r, sl] * red_s[r, sl]
                    return _

                lax.fori_loop(0, SGROUPS, upd_body, None)
                pltpu.sync_copy(red_s, spmem_sum.at[ownrows])

                @pl.when(p == FD_ITERS - 1)
                def _():
                    pltpu.sync_copy(red_s, outx_hbm.at[coord, t])

                plsc.subcore_barrier()
                pltpu.sync_copy(spmem_sum, x_cur)
                return _

            lax.fori_loop(0, FD_ITERS, iter_body, None)

            scatter_pass(with_q_times_x=True)
            reduce_to_slice()
            pltpu.sync_copy(loads_hbm.at[coord, t], b_s)

            def res_body(g, _):
                r, cb = rc(g)
                sl = pl.ds(cb, 16)
                xv = x_cur[r_off + r, sl]
                red_s[r, sl] = (b_s[r, sl] + red_s[r, sl]
                                - diag_s[r, sl] * xv)
                return _

            lax.fori_loop(0, SGROUPS, res_body, None)
            pltpu.sync_copy(red_s, outres_hbm.at[coord, t])

    pltpu.make_async_copy(zeros_hbm, s_part, zsem).wait()


@jax.jit
def _fd_solve(xs, loads, epack, pins, zeros, idrows, ownrows):
    mesh = plsc.VectorSubcoreMesh(core_axis_name="c", subcore_axis_name="s")
    f = pl.kernel(
        _fd_body,
        out_type=(
            jax.ShapeDtypeStruct((3, NTILE, ROWS_T, COLS), jnp.float32),
            jax.ShapeDtypeStruct((3, NTILE, ROWS_T, COLS), jnp.float32),
            jax.ShapeDtypeStruct((2, NTILE, NTILE, ROWS_T, COLS),
                                 jnp.float32),
        ),
        mesh=mesh,
        compiler_params=pltpu.CompilerParams(needs_layout_passes=False),
        scratch_types=[
            pltpu.VMEM((ROWS, COLS), jnp.float32),
            pltpu.VMEM((ROWS, COLS), jnp.float32),
            pltpu.VMEM((2 * ECROWS, COLS), jnp.int32),
            pltpu.VMEM((2 * ECROWS, COLS), jnp.int32),
            pltpu.VMEM((RCHUNKS, RCLEN), jnp.int32),
            pltpu.VMEM((ROWS_T,), jnp.int32),
            pltpu.VMEM((ROWS_T, COLS), jnp.float32),
            pltpu.VMEM((ROWS_T, COLS), jnp.float32),
            pltpu.VMEM((ROWS_T, COLS), jnp.float32),
            pltpu.VMEM((ROWS_T, COLS), jnp.float32),
            pltpu.SemaphoreType.DMA,
            pltpu.SemaphoreType.DMA,
            pltpu.SemaphoreType.DMA,
            pltpu.SemaphoreType.DMA,
            pltpu.VMEM_SHARED((ROWS, COLS), jnp.float32),
        ],
    )
    outx, outres, _parts = f(xs, loads, epack, pins, zeros, idrows, ownrows)
    return outx, outres


N_CE = L * T
N_SP = L * T_SP
NW_AR = T
NW_SP = T_SP


def _iface_body(res_hbm, x_hbm, cl_hbm, corig_hbm, cl2_hbm,
                icem_hbm, ifdm_hbm, icsp_hbm, ifsp_hbm, zi_hbm,
                loads1_hbm, orig1_hbm, loads2_hbm,
                wbuf, clb, idxb, fdmb, rvals, xvals, origb, wtmp, gsem):
    cx = lax.axis_index("c")
    t = lax.axis_index("s")
    lane = jax.lax.iota(jnp.int32, 16)
    neg1 = jnp.full((16,), -1, jnp.int32)

    def job(n_upd, n_dst, icem_src, ifdm_src, cl_src, out_dst,
            with_origin):
        coord = t
        groups = n_upd // 16
        pltpu.sync_copy(icem_src, idxb.at[pl.ds(0, n_upd)])
        pltpu.sync_copy(ifdm_src, fdmb.at[pl.ds(0, n_upd)])
        pltpu.sync_copy(cl_src.at[coord], clb.at[pl.ds(0, n_dst)])
        pltpu.sync_copy(zi_hbm.at[pl.ds(0, n_dst)], wbuf.at[pl.ds(0, n_dst)])
        wtmp[pl.ds(16, 16)] = neg1

        def gidx_body(g, _):
            sl = pl.ds(g * 16, 16)
            fdmb[sl] = fdmb[sl] + coord * (ROWS * COLS)
            return _

        lax.fori_loop(0, groups, gidx_body, None)
        pltpu.async_copy(res_hbm.at[fdmb.at[pl.ds(0, n_upd)]],
                         rvals.at[pl.ds(0, n_upd)], gsem).wait()
        if with_origin:
            pltpu.async_copy(x_hbm.at[fdmb.at[pl.ds(0, n_upd)]],
                             xvals.at[pl.ds(0, n_upd)], gsem).wait()
            pltpu.sync_copy(corig_hbm.at[coord], origb)

        def w_body(g, _):
            sl = pl.ds(g * 16, 16)
            idxv = idxb[sl]
            key = (idxv << 4) | lane
            sk, _sv = plsc.sort_key_val(key, key)
            kk = lax.shift_right_logical(sk, 4)
            slot = (sk & 15) + g * 16
            wtmp[pl.ds(0, 16)] = kk
            endm = kk != wtmp[pl.ds(1, 16)]
            plsc.store_scatter(wbuf, [kk], slot + 1, mask=endm)
            return _

        lax.fori_loop(0, groups, w_body, None)

        def a_body(g, _):
            sl = pl.ds(g * 16, 16)
            idxv = idxb[sl]
            slot = g * 16 + lane
            wv = plsc.load_gather(wbuf, [idxv])
            keep = wv == slot + 1
            plsc.store_scatter(clb, [idxv], rvals[sl], mask=keep)
            if with_origin:
                keep2 = keep & (idxv < NW_AR)
                plsc.store_scatter(origb, [idxv], xvals[sl], mask=keep2)
            return _

        lax.fori_loop(0, groups, a_body, None)
        pltpu.sync_copy(clb.at[pl.ds(0, n_dst)], out_dst.at[coord])
        if with_origin:
            pltpu.sync_copy(origb, orig1_hbm.at[coord])

    @pl.when((cx == 0) & (t < 3))
    def _():
        job(NW_AR, N_CE, icem_hbm, ifdm_hbm, cl_hbm, loads1_hbm, True)

    @pl.when((cx == 1) & (t < 3))
    def _():
        job(NW_SP, N_SP, icsp_hbm, ifsp_hbm, cl2_hbm, loads2_hbm, False)


@jax.jit
def _iface(res_flat, x_flat, cl, corig, cl2, icem, ifdm, icsp, ifsp, zi):
    mesh = plsc.VectorSubcoreMesh(core_axis_name="c", subcore_axis_name="s")
    f = pl.kernel(
        _iface_body,
        out_type=(
            jax.ShapeDtypeStruct((3, N_CE), jnp.float32),
            jax.ShapeDtypeStruct((3, T), jnp.float32),
            jax.ShapeDtypeStruct((3, N_SP), jnp.float32),
        ),
        mesh=mesh,
        compiler_params=pltpu.CompilerParams(needs_layout_passes=False),
        scratch_types=[
            pltpu.VMEM((N_CE,), jnp.int32),
            pltpu.VMEM((N_CE,), jnp.float32),
            pltpu.VMEM((T,), jnp.int32),
            pltpu.VMEM((T,), jnp.int32),
            pltpu.VMEM((T,), jnp.float32),
            pltpu.VMEM((T,), jnp.float32),
            pltpu.VMEM((T,), jnp.float32),
            pltpu.VMEM((32,), jnp.int32),
            pltpu.SemaphoreType.DMA,
        ],
    )
    return f(res_flat, x_flat, cl, corig, cl2, icem, ifdm, icsp, ifsp, zi)


def _ce_body(lx, ly, lz, ln, ox, oy, oz,
             ysx, ysy, ysz, rx, ry, rz):
    init = (ox[0, :], oy[0, :], oz[0, :],
            jnp.zeros_like(ox[0, :]), jnp.zeros_like(ox[0, :]),
            jnp.zeros_like(ox[0, :]))

    def step(s, carry):
        px, py, pz, ex, ey, ez = carry
        ex = ex + lx[s, :]
        ey = ey + ly[s, :]
        ez = ez + lz[s, :]
        scale = ln[s, :] / (jnp.sqrt(ex * ex + ey * ey + ez * ez) + 1e-8)
        px = px + ex * scale
        py = py + ey * scale
        pz = pz + ez * scale
        ysx[s, :] = px
        ysy[s, :] = py
        ysz[s, :] = pz
        return (px, py, pz, ex, ey, ez)

    px, py, pz, ex, ey, ez = lax.fori_loop(0, L, step, init)
    rx[0, :] = ex
    ry[0, :] = ey
    rz[0, :] = ez


def _ce(loads_soa, origin_soa, lengths, n_trails):
    loads = loads_soa.reshape(3, L, n_trails)
    origin = origin_soa
    f = pl.pallas_call(
        _ce_body,
        out_shape=(
            jax.ShapeDtypeStruct((L, n_trails), jnp.float32),
            jax.ShapeDtypeStruct((L, n_trails), jnp.float32),
            jax.ShapeDtypeStruct((L, n_trails), jnp.float32),
            jax.ShapeDtypeStruct((1, n_trails), jnp.float32),
            jax.ShapeDtypeStruct((1, n_trails), jnp.float32),
            jax.ShapeDtypeStruct((1, n_trails), jnp.float32),
        ),
    )
    ysx, ysy, ysz, rx, ry, rz = f(
        loads[0], loads[1], loads[2], lengths,
        origin[0][None, :], origin[1][None, :], origin[2][None, :])
    ys = jnp.stack([ysx, ysy, ysz], axis=-1)
    res_f = jnp.stack([rx[0], ry[0], rz[0]], axis=-1)
    return ys.reshape(L * n_trails, 3), res_f


def kernel(fd_xyz, fd_loads, fd_edges, fd_q, indices_fdm, indices_spoke_fdm,
           indices_cem, indices_spoke_cem, cem_loads, cem_xyz, ce_lengths,
           cem2_loads, cem2_xyz, ce_spoke_lengths):
    xs = jnp.zeros((3, N_PAD), jnp.float32).at[:, :N_FD].set(fd_xyz.T)
    loads = jnp.zeros((3, N_PAD), jnp.float32).at[:, :N_FD].set(fd_loads.T)
    xs = xs.reshape(3, ROWS, COLS)
    loads = loads.reshape(3, NTILE, ROWS_T, COLS)
    ei = jnp.full((E_PAD,), DUMMY, jnp.int32).at[:E_FD].set(fd_edges[0])
    ej = jnp.full((E_PAD,), DUMMY, jnp.int32).at[:E_FD].set(fd_edges[1])
    eqv = jnp.zeros((E_PAD,), jnp.float32).at[:E_FD].set(fd_q)
    packed = (ej << 16) | ei
    pk = packed.reshape(NTILE * NCHUNK, ECROWS, COLS)
    qk = lax.bitcast_convert_type(eqv, jnp.int32).reshape(
        NTILE * NCHUNK, ECROWS, COLS)
    epack = jnp.concatenate([pk, qk], axis=1)
    pins = jnp.concatenate([indices_fdm, indices_spoke_fdm]).astype(jnp.int32)
    pins = jnp.full((N_PIN_PAD,), DUMMY, jnp.int32).at[:N_PIN].set(pins)
    pins = pins.reshape(2, 16, COLS)
    zeros = jnp.zeros((ROWS, COLS), jnp.float32)
    idrows = jnp.arange(ROWS, dtype=jnp.int32).reshape(RCHUNKS, RCLEN)
    ownrows = jnp.arange(ROWS, dtype=jnp.int32).reshape(NTILE, ROWS_T)

    outx, outres = _fd_solve(xs, loads, epack, pins, zeros, idrows, ownrows)
    fd_xyz_out = outx.reshape(3, N_PAD)[:, :N_FD].T
    fd_res = outres.reshape(3, N_PAD)[:, :N_FD].T

    res_flat = outres.reshape(3 * N_PAD)
    x_flat = outx.reshape(3 * N_PAD)
    cl = cem_loads.T
    corig = cem_xyz[:T].T
    cl2 = cem2_loads.T
    zi = jnp.zeros((N_CE,), jnp.int32)
    loads1, orig1, loads2 = _iface(
        res_flat, x_flat, cl, corig, cl2,
        indices_cem.astype(jnp.int32), indices_fdm.astype(jnp.int32),
        indices_spoke_cem.astype(jnp.int32),
        indices_spoke_fdm.astype(jnp.int32), zi)
    ce_xyz, ce_res = _ce(loads1, orig1, ce_lengths, T)
    orig2 = cem2_xyz[:T_SP].T
    spoke_xyz, spoke_res = _ce(loads2, orig2, ce_spoke_lengths, T_SP)
    return (ce_xyz, ce_res, fd_xyz_out, fd_res, spoke_xyz, spoke_res)


# speedup vs baseline: 1.0902x; 1.0902x over previous
import jax
import jax.numpy as jnp
from jax import lax
from jax.experimental import pallas as pl
from jax.experimental.pallas import tpu as pltpu
from jax.experimental.pallas import tpu_sc as plsc

N_FD = 50000
E_FD = 800000
L = 24
T = 2048
T_SP = 1024
FD_ITERS = 8
N_PIN = 3072
N_PIN_PAD = 4096

NTILE = 16
COLS = 128
ROWS = 400
N_PAD = ROWS * COLS
ROWS_T = ROWS // NTILE
SGROUPS = ROWS_T * (COLS // 16)
CHUNK = 1280
ECROWS = CHUNK // COLS
NCHUNK = 40
E_TILE = CHUNK * NCHUNK
E_PAD = NTILE * E_TILE
GROUPS = CHUNK // 16
RCHUNKS = 5
RCLEN = ROWS // RCHUNKS
DUMMY = N_PAD - 1


def _fd_body(xs_hbm, loads_hbm, epack_hbm, pins_hbm, zeros_hbm,
             idrows_hbm, ownrows_hbm, outx_hbm, outres_hbm,
             x_cur, s_part, ebuf0, ebuf1, idrows, ownrows, diag_s, a_s,
             b_s, red_s, esem0, esem1, zsem, rsem, spmem_sum):
    c = lax.axis_index("c")
    t = lax.axis_index("s")
    r_off = t * ROWS_T
    ones16 = jnp.full((16,), 1.0, jnp.float32)

    def rc(g):
        return g // 8, (g % 8) * 16

    def start_chunk(k, eb, sem):
        pltpu.async_copy(epack_hbm.at[t * NCHUNK + k], eb, sem)

    def wait_chunk(k, eb, sem):
        pltpu.make_async_copy(epack_hbm.at[t * NCHUNK + k], eb, sem).wait()

    def compute_chunk(eb, with_q_times_x):
        @plsc.parallel_loop(0, GROUPS, unroll=4)
        def _(g):
            r, cb = rc(g)
            sl = pl.ds(cb, 16)
            v = eb[r, sl]
            qv = plsc.bitcast(eb[ECROWS + r, sl], jnp.float32)
            iv = v & 0xFFFF
            jv = lax.shift_right_logical(v, 16)
            ir, ic = iv >> 7, iv & 127
            jr, jc = jv >> 7, jv & 127
            if with_q_times_x:
                xj = plsc.load_gather(x_cur, [jr, jc])
                xi = plsc.load_gather(x_cur, [ir, ic])
                plsc.addupdate_scatter(s_part, [ir, ic], qv * xj)
                plsc.addupdate_scatter(s_part, [jr, jc], qv * xi)
            else:
                plsc.addupdate_scatter(s_part, [ir, ic], qv)
                plsc.addupdate_scatter(s_part, [jr, jc], qv)

    def start_zero():
        pltpu.async_copy(zeros_hbm, s_part, zsem)

    def scatter_pass(with_q_times_x):
        start_chunk(0, ebuf0, esem0)
        pltpu.make_async_copy(zeros_hbm, s_part, zsem).wait()
        pltpu.sync_copy(s_part.at[pl.ds(0, ROWS_T)], spmem_sum.at[ownrows])

        def pair_body(kk, _):
            k0 = 2 * kk
            start_chunk(k0 + 1, ebuf1, esem1)
            wait_chunk(k0, ebuf0, esem0)
            compute_chunk(ebuf0, with_q_times_x)

            @pl.when(k0 + 2 < NCHUNK)
            def _():
                start_chunk(k0 + 2, ebuf0, esem0)

            wait_chunk(k0 + 1, ebuf1, esem1)
            compute_chunk(ebuf1, with_q_times_x)
            return _

        lax.fori_loop(0, NCHUNK // 2, pair_body, None)

    def reduce_to_slice():
        plsc.subcore_barrier()
        for ch in range(RCHUNKS):
            pltpu.async_copy(s_part.at[pl.ds(ch * RCLEN, RCLEN)],
                             spmem_sum.at[idrows.at[ch]], rsem, add=True)
        for ch in range(RCHUNKS):
            pltpu.make_async_copy(s_part.at[pl.ds(ch * RCLEN, RCLEN)],
                                  spmem_sum.at[idrows.at[ch]], rsem).wait()
        start_zero()
        plsc.subcore_barrier()
        pltpu.sync_copy(spmem_sum.at[ownrows], red_s)

    pltpu.sync_copy(idrows_hbm, idrows)
    pltpu.sync_copy(ownrows_hbm.at[t], ownrows)

    pltpu.sync_copy(zeros_hbm, s_part)
    pltpu.sync_copy(pins_hbm.at[0], ebuf0.at[pl.ds(0, 16)])
    pltpu.sync_copy(pins_hbm.at[1], ebuf1.at[pl.ds(0, 16)])

    for eb in (ebuf0, ebuf1):
        def pin_body(g, _, eb=eb):
            r, cb = rc(g)
            pv = eb[r, pl.ds(cb, 16)]
            plsc.addupdate_scatter(s_part, [pv >> 7, pv & 127], ones16)
            return _

        lax.fori_loop(0, (N_PIN_PAD // 2) // 16, pin_body, None)

    def mask_body(g, _):
        r, cb = rc(g)
        a_s[r, pl.ds(cb, 16)] = 1.0 - jnp.minimum(
            s_part[r_off + r, pl.ds(cb, 16)], 1.0)
        return _

    lax.fori_loop(0, SGROUPS, mask_body, None)
    start_zero()

    scatter_pass(with_q_times_x=False)
    reduce_to_slice()

    def diag_body(g, _):
        r, cb = rc(g)
        sl = pl.ds(cb, 16)
        dv = red_s[r, sl] + 1e-6
        diag_s[r, sl] = dv
        a_s[r, sl] = a_s[r, sl] / dv
        return _

    lax.fori_loop(0, SGROUPS, diag_body, None)
    plsc.subcore_barrier()

    for slot in range(2):
        coord = 2 * c + slot

        @pl.when(coord <= 2)
        def _():
            pltpu.sync_copy(xs_hbm.at[coord], x_cur)
            pltpu.sync_copy(loads_hbm.at[coord, t], b_s)

            def b_body(g, _):
                r, cb = rc(g)
                sl = pl.ds(cb, 16)
                m = 1.0 - a_s[r, sl] * diag_s[r, sl]
                x0 = x_cur[r_off + r, sl]
                b_s[r, sl] = m * x0 + b_s[r, sl] * a_s[r, sl]
                return _

            lax.fori_loop(0, SGROUPS, b_body, None)
            plsc.subcore_barrier()

            def iter_body(p, _):
                scatter_pass(with_q_times_x=True)
                reduce_to_slice()

                def upd_body(g, _):
                    r, cb = rc(g)
                    sl = pl.ds(cb, 16)
                    red_s[r, sl] = b_s[r, sl] + a_s[r, sl] * red_s[r, sl]
                    return _

                lax.fori_loop(0, SGROUPS, upd_body, None)
                pltpu.sync_copy(red_s, spmem_sum.at[ownrows])

                @pl.when(p == FD_ITERS - 1)
                def _():
                    pltpu.sync_copy(red_s, outx_hbm.at[coord, t])

                plsc.subcore_barrier()
                pltpu.sync_copy(spmem_sum, x_cur)
                plsc.subcore_barrier()
                return _

            lax.fori_loop(0, FD_ITERS, iter_body, None)

            scatter_pass(with_q_times_x=True)
            reduce_to_slice()
            pltpu.sync_copy(loads_hbm.at[coord, t], b_s)

            def res_body(g, _):
                r, cb = rc(g)
                sl = pl.ds(cb, 16)
                xv = x_cur[r_off + r, sl]
                red_s[r, sl] = (b_s[r, sl] + red_s[r, sl]
                                - diag_s[r, sl] * xv)
                return _

            lax.fori_loop(0, SGROUPS, res_body, None)
            pltpu.sync_copy(red_s, outres_hbm.at[coord, t])

    pltpu.make_async_copy(zeros_hbm, s_part, zsem).wait()


@jax.jit
def _fd_solve(xs, loads, epack, pins, zeros, idrows, ownrows):
    mesh = plsc.VectorSubcoreMesh(core_axis_name="c", subcore_axis_name="s")
    f = pl.kernel(
        _fd_body,
        out_type=(
            jax.ShapeDtypeStruct((3, NTILE, ROWS_T, COLS), jnp.float32),
            jax.ShapeDtypeStruct((3, NTILE, ROWS_T, COLS), jnp.float32),
        ),
        mesh=mesh,
        compiler_params=pltpu.CompilerParams(needs_layout_passes=False),
        scratch_types=[
            pltpu.VMEM((ROWS, COLS), jnp.float32),
            pltpu.VMEM((ROWS, COLS), jnp.float32),
            pltpu.VMEM((2 * ECROWS, COLS), jnp.int32),
            pltpu.VMEM((2 * ECROWS, COLS), jnp.int32),
            pltpu.VMEM((RCHUNKS, RCLEN), jnp.int32),
            pltpu.VMEM((ROWS_T,), jnp.int32),
            pltpu.VMEM((ROWS_T, COLS), jnp.float32),
            pltpu.VMEM((ROWS_T, COLS), jnp.float32),
            pltpu.VMEM((ROWS_T, COLS), jnp.float32),
            pltpu.VMEM((ROWS_T, COLS), jnp.float32),
            pltpu.SemaphoreType.DMA,
            pltpu.SemaphoreType.DMA,
            pltpu.SemaphoreType.DMA,
            pltpu.SemaphoreType.DMA,
            pltpu.VMEM_SHARED((ROWS, COLS), jnp.float32),
        ],
    )
    return f(xs, loads, epack, pins, zeros, idrows, ownrows)


N_CE = L * T
N_SP = L * T_SP
NW_AR = T
NW_SP = T_SP


def _iface_body(res_hbm, x_hbm, cl_hbm, corig_hbm, cl2_hbm,
                icem_hbm, ifdm_hbm, icsp_hbm, ifsp_hbm, zi_hbm,
                loads1_hbm, orig1_hbm, loads2_hbm,
                wbuf, clb, idxb, fdmb, rvals, xvals, origb, wtmp, gsem):
    cx = lax.axis_index("c")
    t = lax.axis_index("s")
    lane = jax.lax.iota(jnp.int32, 16)
    neg1 = jnp.full((16,), -1, jnp.int32)

    def job(n_upd, n_dst, icem_src, ifdm_src, cl_src, out_dst,
            with_origin):
        coord = t
        groups = n_upd // 16
        pltpu.sync_copy(icem_src, idxb.at[pl.ds(0, n_upd)])
        pltpu.sync_copy(ifdm_src, fdmb.at[pl.ds(0, n_upd)])
        pltpu.sync_copy(cl_src.at[coord], clb.at[pl.ds(0, n_dst)])
        pltpu.sync_copy(zi_hbm.at[pl.ds(0, n_dst)], wbuf.at[pl.ds(0, n_dst)])
        wtmp[pl.ds(16, 16)] = neg1

        def gidx_body(g, _):
            sl = pl.ds(g * 16, 16)
            fdmb[sl] = fdmb[sl] + coord * (ROWS * COLS)
            return _

        lax.fori_loop(0, groups, gidx_body, None)
        pltpu.async_copy(res_hbm.at[fdmb.at[pl.ds(0, n_upd)]],
                         rvals.at[pl.ds(0, n_upd)], gsem).wait()
        if with_origin:
            pltpu.async_copy(x_hbm.at[fdmb.at[pl.ds(0, n_upd)]],
                             xvals.at[pl.ds(0, n_upd)], gsem).wait()
            pltpu.sync_copy(corig_hbm.at[coord], origb)

        def w_body(g, _):
            sl = pl.ds(g * 16, 16)
            idxv = idxb[sl]
            key = (idxv << 4) | lane
            sk, _sv = plsc.sort_key_val(key, key)
            kk = lax.shift_right_logical(sk, 4)
            slot = (sk & 15) + g * 16
            wtmp[pl.ds(0, 16)] = kk
            endm = kk != wtmp[pl.ds(1, 16)]
            plsc.store_scatter(wbuf, [kk], slot + 1, mask=endm)
            return _

        lax.fori_loop(0, groups, w_body, None)

        def a_body(g, _):
            sl = pl.ds(g * 16, 16)
            idxv = idxb[sl]
            slot = g * 16 + lane
            wv = plsc.load_gather(wbuf, [idxv])
            keep = wv == slot + 1
            plsc.store_scatter(clb, [idxv], rvals[sl], mask=keep)
            if with_origin:
                keep2 = keep & (idxv < NW_AR)
                plsc.store_scatter(origb, [idxv], xvals[sl], mask=keep2)
            return _

        lax.fori_loop(0, groups, a_body, None)
        pltpu.sync_copy(clb.at[pl.ds(0, n_dst)], out_dst.at[coord])
        if with_origin:
            pltpu.sync_copy(origb, orig1_hbm.at[coord])

    @pl.when((cx == 0) & (t < 3))
    def _():
        job(NW_AR, N_CE, icem_hbm, ifdm_hbm, cl_hbm, loads1_hbm, True)

    @pl.when((cx == 1) & (t < 3))
    def _():
        job(NW_SP, N_SP, icsp_hbm, ifsp_hbm, cl2_hbm, loads2_hbm, False)


@jax.jit
def _iface(res_flat, x_flat, cl, corig, cl2, icem, ifdm, icsp, ifsp, zi):
    mesh = plsc.VectorSubcoreMesh(core_axis_name="c", subcore_axis_name="s")
    f = pl.kernel(
        _iface_body,
        out_type=(
            jax.ShapeDtypeStruct((3, N_CE), jnp.float32),
            jax.ShapeDtypeStruct((3, T), jnp.float32),
            jax.ShapeDtypeStruct((3, N_SP), jnp.float32),
        ),
        mesh=mesh,
        compiler_params=pltpu.CompilerParams(needs_layout_passes=False),
        scratch_types=[
            pltpu.VMEM((N_CE,), jnp.int32),
            pltpu.VMEM((N_CE,), jnp.float32),
            pltpu.VMEM((T,), jnp.int32),
            pltpu.VMEM((T,), jnp.int32),
            pltpu.VMEM((T,), jnp.float32),
            pltpu.VMEM((T,), jnp.float32),
            pltpu.VMEM((T,), jnp.float32),
            pltpu.VMEM((32,), jnp.int32),
            pltpu.SemaphoreType.DMA,
        ],
    )
    return f(res_flat, x_flat, cl, corig, cl2, icem, ifdm, icsp, ifsp, zi)


def _ce_body(lx, ly, lz, ln, ox, oy, oz,
             ysx, ysy, ysz, rx, ry, rz):
    init = (ox[0, :], oy[0, :], oz[0, :],
            jnp.zeros_like(ox[0, :]), jnp.zeros_like(ox[0, :]),
            jnp.zeros_like(ox[0, :]))

    def step(s, carry):
        px, py, pz, ex, ey, ez = carry
        ex = ex + lx[s, :]
        ey = ey + ly[s, :]
        ez = ez + lz[s, :]
        scale = ln[s, :] / (jnp.sqrt(ex * ex + ey * ey + ez * ez) + 1e-8)
        px = px + ex * scale
        py = py + ey * scale
        pz = pz + ez * scale
        ysx[s, :] = px
        ysy[s, :] = py
        ysz[s, :] = pz
        return (px, py, pz, ex, ey, ez)

    px, py, pz, ex, ey, ez = lax.fori_loop(0, L, step, init)
    rx[0, :] = ex
    ry[0, :] = ey
    rz[0, :] = ez


def _ce(loads_soa, origin_soa, lengths, n_trails):
    loads = loads_soa.reshape(3, L, n_trails)
    origin = origin_soa
    f = pl.pallas_call(
        _ce_body,
        out_shape=(
            jax.ShapeDtypeStruct((L, n_trails), jnp.float32),
            jax.ShapeDtypeStruct((L, n_trails), jnp.float32),
            jax.ShapeDtypeStruct((L, n_trails), jnp.float32),
            jax.ShapeDtypeStruct((1, n_trails), jnp.float32),
            jax.ShapeDtypeStruct((1, n_trails), jnp.float32),
            jax.ShapeDtypeStruct((1, n_trails), jnp.float32),
        ),
    )
    ysx, ysy, ysz, rx, ry, rz = f(
        loads[0], loads[1], loads[2], lengths,
        origin[0][None, :], origin[1][None, :], origin[2][None, :])
    ys = jnp.stack([ysx, ysy, ysz], axis=-1)
    res_f = jnp.stack([rx[0], ry[0], rz[0]], axis=-1)
    return ys.reshape(L * n_trails, 3), res_f


def kernel(fd_xyz, fd_loads, fd_edges, fd_q, indices_fdm, indices_spoke_fdm,
           indices_cem, indices_spoke_cem, cem_loads, cem_xyz, ce_lengths,
           cem2_loads, cem2_xyz, ce_spoke_lengths):
    xs = jnp.zeros((3, N_PAD), jnp.float32).at[:, :N_FD].set(fd_xyz.T)
    loads = jnp.zeros((3, N_PAD), jnp.float32).at[:, :N_FD].set(fd_loads.T)
    xs = xs.reshape(3, ROWS, COLS)
    loads = loads.reshape(3, NTILE, ROWS_T, COLS)
    ei = jnp.full((E_PAD,), DUMMY, jnp.int32).at[:E_FD].set(fd_edges[0])
    ej = jnp.full((E_PAD,), DUMMY, jnp.int32).at[:E_FD].set(fd_edges[1])
    eqv = jnp.zeros((E_PAD,), jnp.float32).at[:E_FD].set(fd_q)
    packed = (ej << 16) | ei
    pk = packed.reshape(NTILE * NCHUNK, ECROWS, COLS)
    qk = lax.bitcast_convert_type(eqv, jnp.int32).reshape(
        NTILE * NCHUNK, ECROWS, COLS)
    epack = jnp.concatenate([pk, qk], axis=1)
    pins = jnp.concatenate([indices_fdm, indices_spoke_fdm]).astype(jnp.int32)
    pins = jnp.full((N_PIN_PAD,), DUMMY, jnp.int32).at[:N_PIN].set(pins)
    pins = pins.reshape(2, 16, COLS)
    zeros = jnp.zeros((ROWS, COLS), jnp.float32)
    idrows = jnp.arange(ROWS, dtype=jnp.int32).reshape(RCHUNKS, RCLEN)
    ownrows = jnp.arange(ROWS, dtype=jnp.int32).reshape(NTILE, ROWS_T)

    outx, outres = _fd_solve(xs, loads, epack, pins, zeros, idrows, ownrows)
    fd_xyz_out = outx.reshape(3, N_PAD)[:, :N_FD].T
    fd_res = outres.reshape(3, N_PAD)[:, :N_FD].T

    res_flat = outres.reshape(3 * N_PAD)
    x_flat = outx.reshape(3 * N_PAD)
    cl = cem_loads.T
    corig = cem_xyz[:T].T
    cl2 = cem2_loads.T
    zi = jnp.zeros((N_CE,), jnp.int32)
    loads1, orig1, loads2 = _iface(
        res_flat, x_flat, cl, corig, cl2,
        indices_cem.astype(jnp.int32), indices_fdm.astype(jnp.int32),
        indices_spoke_cem.astype(jnp.int32),
        indices_spoke_fdm.astype(jnp.int32), zi)
    ce_xyz, ce_res = _ce(loads1, orig1, ce_lengths, T)
    orig2 = cem2_xyz[:T_SP].T
    spoke_xyz, spoke_res = _ce(loads2, orig2, ce_spoke_lengths, T_SP)
    return (ce_xyz, ce_res, fd_xyz_out, fd_res, spoke_xyz, spoke_res)
